# Initial kernel scaffold; baseline (speedup 1.0000x reference)
import jax
import jax.numpy as jnp
from jax.experimental import pallas as pl


def kernel(x, ln1_g, ln1_b, ln2_g, ln2_b, Wq, Wk, Wv, Wo, Wsv, Wso, Wg, Wm, bm):
    raise NotImplementedError("write your pallas kernel here")


# speedup vs baseline: 1.4490x; 1.4490x over previous
import jax
import jax.numpy as jnp
from jax.experimental import pallas as pl
from jax.experimental.pallas import tpu as pltpu

_DIM = 768
_H = 12
_DH = 64
_E = 6
_S = 2048

_SB = 256
_SQ = 512

_bf16 = jnp.bfloat16
_f32 = jnp.float32


def _head_expand_matrix():
  rows = jax.lax.broadcasted_iota(jnp.int32, (_H, _DIM), 0)
  cols = jax.lax.broadcasted_iota(jnp.int32, (_H, _DIM), 1)
  return (cols // _DH == rows).astype(_f32)


def _top1_sigmoid(scores_em):
  best_s = jnp.full((scores_em.shape[0], _H), -jnp.inf, _f32)
  best_i = jnp.zeros((scores_em.shape[0], _H), jnp.int32)
  for e in range(_E):
    se = scores_em[:, e * _H:(e + 1) * _H]
    upd = se > best_s
    best_s = jnp.where(upd, se, best_s)
    best_i = jnp.where(upd, e, best_i)
  return jax.nn.sigmoid(best_s), best_i


def _layernorm(xb, g, b):
  m = jnp.mean(xb, axis=-1, keepdims=True)
  xc = xb - m
  v = jnp.mean(xc * xc, axis=-1, keepdims=True)
  return xc / jnp.sqrt(v + 1e-5) * g + b


def _pre_body(x_ref, g_ref, b_ref, wq_ref, wk_ref, wsv_ref, wso_ref, wv_ref,
              q_ref, k_ref, v_ref, mow_ref, moi_ref):
  xn = _layernorm(x_ref[...], g_ref[...], b_ref[...])
  xnb = xn.astype(_bf16)
  q_ref[...] = jnp.dot(xnb, wq_ref[...], preferred_element_type=_f32).astype(_bf16)
  k_ref[...] = jnp.dot(xnb, wk_ref[...], preferred_element_type=_f32).astype(_bf16)
  sv = jnp.dot(xnb, wsv_ref[...], preferred_element_type=_f32)
  so = jnp.dot(xnb, wso_ref[...], preferred_element_type=_f32)
  vw, vi = _top1_sigmoid(sv)
  ow, oi = _top1_sigmoid(so)
  mow_ref[...] = ow
  moi_ref[...] = oi
  rexp = _head_expand_matrix()
  vw_b = vw.astype(_bf16).astype(_f32)
  vacc = jnp.zeros((x_ref.shape[0], _DIM), _f32)
  for e in range(_E):
    ve = jnp.dot(xnb, wv_ref[e], preferred_element_type=_f32)
    ve_b = ve.astype(_bf16).astype(_f32)
    m768 = jnp.dot(jnp.where(vi == e, vw_b, 0.0), rexp,
                   preferred_element_type=_f32)
    vacc += m768 * ve_b
  v_ref[...] = vacc.astype(_bf16)


def _attn_body(q_ref, k_ref, v_ref, o_ref):
  scale = 1.0 / jnp.sqrt(jnp.float32(_DH))
  outs = []
  for h in range(_H):
    sl = slice(h * _DH, (h + 1) * _DH)
    qh = q_ref[:, sl]
    kh = k_ref[:, sl]
    vh = v_ref[:, sl]
    s = jax.lax.dot_general(qh, kh, (((1,), (1,)), ((), ())),
                            preferred_element_type=_f32) * scale
    m = jnp.max(s, axis=-1, keepdims=True)
    p = jnp.exp(s - m)
    l = jnp.sum(p, axis=-1, keepdims=True)
    pn = (p / l).astype(_bf16)
    o = jax.lax.dot_general(pn, vh, (((1,), (0,)), ((), ())),
                            preferred_element_type=_f32)
    outs.append(o.astype(_bf16))
  o_ref[...] = jnp.concatenate(outs, axis=1)


def _post_body(x_ref, o_ref, mow_ref, moi_ref, wo_ref, g2_ref, b2_ref,
               wg_ref, wm_ref, bm_ref, xout_ref):
  rexp = _head_expand_matrix()
  of = o_ref[...].astype(_f32)
  ow_b = mow_ref[...].astype(_bf16).astype(_f32)
  oi = moi_ref[...]
  acc = jnp.zeros((x_ref.shape[0], _DIM), _f32)
  for e in range(_E):
    m768 = jnp.dot(jnp.where(oi == e, ow_b, 0.0), rexp,
                   preferred_element_type=_f32)
    oe = (of * m768).astype(_bf16)
    acc += jnp.dot(oe, wo_ref[e], preferred_element_type=_f32)
  x1 = x_ref[...] + acc
  xn2 = _layernorm(x1, g2_ref[...], b2_ref[...])
  xn2b = xn2.astype(_bf16)
  logits = jnp.dot(xn2b, wg_ref[...], preferred_element_type=_f32)
  lm = jnp.max(logits, axis=-1, keepdims=True)
  ex = jnp.exp(logits - lm)
  gw = 1.0 / jnp.sum(ex, axis=-1, keepdims=True)
  gb = jnp.full((x_ref.shape[0], 1), -jnp.inf, _f32)
  gi = jnp.zeros((x_ref.shape[0], 1), jnp.int32)
  for e in range(_E):
    ge = logits[:, e:e + 1]
    upd = ge > gb
    gb = jnp.where(upd, ge, gb)
    gi = jnp.where(upd, e, gi)
  xn2_b32 = xn2b.astype(_f32)
  gw_b = gw.astype(_bf16).astype(_f32)
  bmf = bm_ref[...].astype(_f32)
  facc = jnp.zeros((x_ref.shape[0], _DIM), _f32)
  for e in range(_E):
    w_e = jnp.where(gi == e, gw_b, 0.0)
    fin = (xn2_b32 * w_e).astype(_bf16)
    fe = jnp.dot(fin, wm_ref[e], preferred_element_type=_f32)
    facc += fe + w_e * bmf[e:e + 1, :]
  xout_ref[...] = x1 + facc


def _const_spec(shape):
  nd = len(shape)
  return pl.BlockSpec(shape, lambda *_: (0,) * nd)


def _run_layer(xb, g1, b1, g2, b2, wq, wk, wsv, wso, wv, wo, wg, wm, bm):
  nblk = _S // _SB
  q, k, v, mow, moi = pl.pallas_call(
      _pre_body,
      grid=(nblk,),
      in_specs=[
          pl.BlockSpec((_SB, _DIM), lambda i: (i, 0)),
          _const_spec((1, _DIM)), _const_spec((1, _DIM)),
          _const_spec((_DIM, _DIM)), _const_spec((_DIM, _DIM)),
          _const_spec((_DIM, _E * _H)), _const_spec((_DIM, _E * _H)),
          _const_spec((_E, _DIM, _DIM)),
      ],
      out_specs=[
          pl.BlockSpec((_SB, _DIM), lambda i: (i, 0)),
          pl.BlockSpec((_SB, _DIM), lambda i: (i, 0)),
          pl.BlockSpec((_SB, _DIM), lambda i: (i, 0)),
          pl.BlockSpec((_SB, _H), lambda i: (i, 0)),
          pl.BlockSpec((_SB, _H), lambda i: (i, 0)),
      ],
      out_shape=[
          jax.ShapeDtypeStruct((_S, _DIM), _bf16),
          jax.ShapeDtypeStruct((_S, _DIM), _bf16),
          jax.ShapeDtypeStruct((_S, _DIM), _bf16),
          jax.ShapeDtypeStruct((_S, _H), _f32),
          jax.ShapeDtypeStruct((_S, _H), jnp.int32),
      ],
      compiler_params=pltpu.CompilerParams(
          dimension_semantics=("arbitrary",)),
  )(xb, g1, b1, wq, wk, wsv, wso, wv)

  o = pl.pallas_call(
      _attn_body,
      grid=(_S // _SQ,),
      in_specs=[
          pl.BlockSpec((_SQ, _DIM), lambda i: (i, 0)),
          _const_spec((_S, _DIM)),
          _const_spec((_S, _DIM)),
      ],
      out_specs=pl.BlockSpec((_SQ, _DIM), lambda i: (i, 0)),
      out_shape=jax.ShapeDtypeStruct((_S, _DIM), _bf16),
      compiler_params=pltpu.CompilerParams(
          dimension_semantics=("arbitrary",)),
  )(q, k, v)

  xout = pl.pallas_call(
      _post_body,
      grid=(nblk,),
      in_specs=[
          pl.BlockSpec((_SB, _DIM), lambda i: (i, 0)),
          pl.BlockSpec((_SB, _DIM), lambda i: (i, 0)),
          pl.BlockSpec((_SB, _H), lambda i: (i, 0)),
          pl.BlockSpec((_SB, _H), lambda i: (i, 0)),
          _const_spec((_E, _DIM, _DIM)),
          _const_spec((1, _DIM)), _const_spec((1, _DIM)),
          _const_spec((_DIM, _E)),
          _const_spec((_E, _DIM, _DIM)),
          _const_spec((_E, _DIM)),
      ],
      out_specs=pl.BlockSpec((_SB, _DIM), lambda i: (i, 0)),
      out_shape=jax.ShapeDtypeStruct((_S, _DIM), _f32),
      compiler_params=pltpu.CompilerParams(
          dimension_semantics=("arbitrary",)),
  )(xb, o, mow, moi, wo, g2, b2, wg, wm, bm)
  return xout


def kernel(x, ln1_g, ln1_b, ln2_g, ln2_b, Wq, Wk, Wv, Wo, Wsv, Wso, Wg, Wm, bm):
  xb = x[0]
  for l in range(2):
    xb = _run_layer(
        xb,
        ln1_g[l][None, :], ln1_b[l][None, :],
        ln2_g[l][None, :], ln2_b[l][None, :],
        Wq[l].astype(_bf16), Wk[l].astype(_bf16),
        Wsv[l].transpose(0, 2, 1).reshape(_DIM, _E * _H).astype(_bf16),
        Wso[l].transpose(0, 2, 1).reshape(_DIM, _E * _H).astype(_bf16),
        Wv[l].reshape(_E, _DIM, _DIM).astype(_bf16),
        Wo[l].reshape(_E, _DIM, _DIM).astype(_bf16),
        Wg[l].astype(_bf16),
        Wm[l].astype(_bf16),
        bm[l].astype(_bf16),
    )
  return xb[None]
